# Initial kernel scaffold; baseline (speedup 1.0000x reference)
import jax
import jax.numpy as jnp
from jax.experimental import pallas as pl


def kernel(R, offsets, idx_i, idx_j):
    raise NotImplementedError("write your pallas kernel here")


# speedup vs baseline: 3.8664x; 3.8664x over previous
import functools

import jax
import jax.numpy as jnp
from jax import lax
from jax.experimental import pallas as pl
from jax.experimental.pallas import tpu as pltpu
from jax.experimental.pallas import tpu_sc as plsc

NC = 2
NS = 16
NW = NC * NS

C = 4000


def _gather_sc(T8, idx_i, idx_j):
    E = idx_i.shape[0]
    per_w = E // NW
    n_chunks = per_w // C
    mesh = plsc.VectorSubcoreMesh(core_axis_name="c", subcore_axis_name="s")

    @functools.partial(
        pl.kernel,
        mesh=mesh,
        compiler_params=pltpu.CompilerParams(use_tc_tiling_on_sc=False),
        out_type=(
            jax.ShapeDtypeStruct((E, 8), jnp.float32),
            jax.ShapeDtypeStruct((E, 8), jnp.float32),
        ),
        scratch_types=[
            pltpu.VMEM((C,), jnp.int32),
            pltpu.VMEM((C,), jnp.int32),
            pltpu.VMEM((C, 8), jnp.float32),
            pltpu.VMEM((C, 8), jnp.float32),
            pltpu.SemaphoreType.DMA,
        ],
    )
    def k(r_hbm, ii_hbm, jj_hbm, oi_hbm, oj_hbm, ii_v, jj_v, pi_v, pj_v, sem):
        wid = lax.axis_index("s") * NC + lax.axis_index("c")

        def chunk_body(t, carry):
            base = wid * per_w + t * C
            pltpu.sync_copy(ii_hbm.at[pl.ds(base, C)], ii_v)
            pltpu.sync_copy(jj_hbm.at[pl.ds(base, C)], jj_v)
            pltpu.async_copy(r_hbm.at[ii_v], pi_v, sem).wait()
            pltpu.async_copy(r_hbm.at[jj_v], pj_v, sem).wait()
            pltpu.sync_copy(pi_v, oi_hbm.at[pl.ds(base, C)])
            pltpu.sync_copy(pj_v, oj_hbm.at[pl.ds(base, C)])
            return carry

        lax.fori_loop(0, n_chunks, chunk_body, 0)

    return k(T8, idx_i, idx_j)


def _combine_body(pi_ref, pj_ref, off_ref, o_ref):
    o_ref[...] = pj_ref[...] - pi_ref[...] + off_ref[...]


def _combine_tc(pi, pj, off):
    rows = off.shape[0]
    br = 5000
    grid = rows // br
    spec = pl.BlockSpec((br, 128), lambda i: (i, 0))
    return pl.pallas_call(
        _combine_body,
        out_shape=jax.ShapeDtypeStruct((rows, 128), jnp.float32),
        grid=(grid,),
        in_specs=[spec, spec, spec],
        out_specs=spec,
    )(pi, pj, off)


def kernel(R, offsets, idx_i, idx_j):
    E = idx_i.shape[0]
    t8 = jnp.pad(R, ((0, 0), (0, 5)))
    pi8, pj8 = _gather_sc(t8, idx_i.astype(jnp.int32), idx_j.astype(jnp.int32))
    off8 = jnp.pad(offsets, ((0, 0), (0, 5)))
    rows = (E * 8) // 128
    out = _combine_tc(pi8.reshape(rows, 128), pj8.reshape(rows, 128),
                      off8.reshape(rows, 128))
    return out.reshape(E, 8)[:, :3]


# speedup vs baseline: 75.0692x; 19.4159x over previous
import functools

import jax
import jax.numpy as jnp
from jax import lax
from jax.experimental import pallas as pl
from jax.experimental.pallas import tpu as pltpu
from jax.experimental.pallas import tpu_sc as plsc

NC = 2
NS = 16
NW = NC * NS

C = 4000
STAGE = 10000


def _gather_sc(rx, ry, rz, idx_i, idx_j):
    E = idx_i.shape[0]
    n = rx.shape[0]
    per_w = E // NW
    n_chunks = per_w // C
    mesh = plsc.VectorSubcoreMesh(core_axis_name="c", subcore_axis_name="s")

    @functools.partial(
        pl.kernel,
        mesh=mesh,
        compiler_params=pltpu.CompilerParams(use_tc_tiling_on_sc=False),
        out_type=tuple(
            jax.ShapeDtypeStruct((E,), jnp.float32) for _ in range(6)
        ),
        scratch_types=[
            pltpu.VMEM((C,), jnp.int32),
            pltpu.VMEM((C,), jnp.int32),
            pltpu.VMEM((C,), jnp.float32),
            pltpu.VMEM((C,), jnp.float32),
            pltpu.VMEM((C,), jnp.float32),
            pltpu.VMEM((C,), jnp.float32),
            pltpu.VMEM((C,), jnp.float32),
            pltpu.VMEM((C,), jnp.float32),
            pltpu.VMEM_SHARED((n,), jnp.float32),
            pltpu.VMEM_SHARED((n,), jnp.float32),
            pltpu.VMEM_SHARED((n,), jnp.float32),
            pltpu.SemaphoreType.DMA,
        ],
    )
    def k(rx_h, ry_h, rz_h, ii_h, jj_h,
          ix_h, iy_h, iz_h, jx_h, jy_h, jz_h,
          ii_v, jj_v, gix, giy, giz, gjx, gjy, gjz,
          rx_sp, ry_sp, rz_sp, sem):
        sid = lax.axis_index("s")
        wid = sid * NC + lax.axis_index("c")

        @pl.when(sid < n // STAGE)
        def _():
            sbase = sid * STAGE
            pltpu.sync_copy(rx_h.at[pl.ds(sbase, STAGE)],
                            rx_sp.at[pl.ds(sbase, STAGE)])
            pltpu.sync_copy(ry_h.at[pl.ds(sbase, STAGE)],
                            ry_sp.at[pl.ds(sbase, STAGE)])
            pltpu.sync_copy(rz_h.at[pl.ds(sbase, STAGE)],
                            rz_sp.at[pl.ds(sbase, STAGE)])

        plsc.subcore_barrier()

        def chunk_body(t, carry):
            base = wid * per_w + t * C
            pltpu.sync_copy(ii_h.at[pl.ds(base, C)], ii_v)
            pltpu.sync_copy(jj_h.at[pl.ds(base, C)], jj_v)
            cps = [
                pltpu.async_copy(rx_sp.at[ii_v], gix, sem),
                pltpu.async_copy(ry_sp.at[ii_v], giy, sem),
                pltpu.async_copy(rz_sp.at[ii_v], giz, sem),
                pltpu.async_copy(rx_sp.at[jj_v], gjx, sem),
                pltpu.async_copy(ry_sp.at[jj_v], gjy, sem),
                pltpu.async_copy(rz_sp.at[jj_v], gjz, sem),
            ]
            for cp in cps:
                cp.wait()
            pltpu.sync_copy(gix, ix_h.at[pl.ds(base, C)])
            pltpu.sync_copy(giy, iy_h.at[pl.ds(base, C)])
            pltpu.sync_copy(giz, iz_h.at[pl.ds(base, C)])
            pltpu.sync_copy(gjx, jx_h.at[pl.ds(base, C)])
            pltpu.sync_copy(gjy, jy_h.at[pl.ds(base, C)])
            pltpu.sync_copy(gjz, jz_h.at[pl.ds(base, C)])
            return carry

        lax.fori_loop(0, n_chunks, chunk_body, 0)

    return k(rx, ry, rz, idx_i, idx_j)


def _combine_body(ax_r, ay_r, az_r, bx_r, by_r, bz_r, ox_r, oy_r, oz_r,
                  cx_r, cy_r, cz_r):
    cx_r[...] = bx_r[...] - ax_r[...] + ox_r[...]
    cy_r[...] = by_r[...] - ay_r[...] + oy_r[...]
    cz_r[...] = bz_r[...] - az_r[...] + oz_r[...]


def _combine_tc(pi, pj, off):
    rows = pi[0].shape[0]
    br = 2000
    grid = rows // br
    spec = pl.BlockSpec((br, 128), lambda i: (i, 0))
    one = jax.ShapeDtypeStruct((rows, 128), jnp.float32)
    return pl.pallas_call(
        _combine_body,
        out_shape=(one, one, one),
        grid=(grid,),
        in_specs=[spec] * 9,
        out_specs=(spec, spec, spec),
    )(*pi, *pj, *off)


def kernel(R, offsets, idx_i, idx_j):
    E = idx_i.shape[0]
    rows = E // 128
    ix, iy, iz, jx, jy, jz = _gather_sc(
        R[:, 0], R[:, 1], R[:, 2],
        idx_i.astype(jnp.int32), idx_j.astype(jnp.int32))
    pi = [a.reshape(rows, 128) for a in (ix, iy, iz)]
    pj = [a.reshape(rows, 128) for a in (jx, jy, jz)]
    off = [offsets[:, c].reshape(rows, 128) for c in range(3)]
    cx, cy, cz = _combine_tc(pi, pj, off)
    return jnp.stack([cx.reshape(E), cy.reshape(E), cz.reshape(E)], axis=1)


# speedup vs baseline: 80.4993x; 1.0723x over previous
import functools

import jax
import jax.numpy as jnp
from jax import lax
from jax.experimental import pallas as pl
from jax.experimental.pallas import tpu as pltpu
from jax.experimental.pallas import tpu_sc as plsc

NC = 2
NS = 16
NW = NC * NS

C = 8000
STAGE = 10000


def _gather_sc(rx, ry, rz, idx_i, idx_j):
    E = idx_i.shape[0]
    n = rx.shape[0]
    per_w = E // NW
    n_chunks = per_w // C
    mesh = plsc.VectorSubcoreMesh(core_axis_name="c", subcore_axis_name="s")

    @functools.partial(
        pl.kernel,
        mesh=mesh,
        compiler_params=pltpu.CompilerParams(use_tc_tiling_on_sc=False),
        out_type=tuple(
            jax.ShapeDtypeStruct((E,), jnp.float32) for _ in range(6)
        ),
        scratch_types=[
            pltpu.VMEM((C,), jnp.int32),
            pltpu.VMEM((C,), jnp.int32),
            pltpu.VMEM((C,), jnp.float32),
            pltpu.VMEM((C,), jnp.float32),
            pltpu.VMEM((C,), jnp.float32),
            pltpu.VMEM((C,), jnp.float32),
            pltpu.VMEM((C,), jnp.float32),
            pltpu.VMEM((C,), jnp.float32),
            pltpu.VMEM_SHARED((n,), jnp.float32),
            pltpu.VMEM_SHARED((n,), jnp.float32),
            pltpu.VMEM_SHARED((n,), jnp.float32),
            pltpu.SemaphoreType.DMA,
        ],
    )
    def k(rx_h, ry_h, rz_h, ii_h, jj_h,
          ix_h, iy_h, iz_h, jx_h, jy_h, jz_h,
          ii_v, jj_v, gix, giy, giz, gjx, gjy, gjz,
          rx_sp, ry_sp, rz_sp, sem):
        sid = lax.axis_index("s")
        wid = sid * NC + lax.axis_index("c")

        @pl.when(sid < n // STAGE)
        def _():
            sbase = sid * STAGE
            pltpu.sync_copy(rx_h.at[pl.ds(sbase, STAGE)],
                            rx_sp.at[pl.ds(sbase, STAGE)])
            pltpu.sync_copy(ry_h.at[pl.ds(sbase, STAGE)],
                            ry_sp.at[pl.ds(sbase, STAGE)])
            pltpu.sync_copy(rz_h.at[pl.ds(sbase, STAGE)],
                            rz_sp.at[pl.ds(sbase, STAGE)])

        plsc.subcore_barrier()

        def chunk_body(t, carry):
            base = wid * per_w + t * C
            pltpu.sync_copy(ii_h.at[pl.ds(base, C)], ii_v)
            pltpu.sync_copy(jj_h.at[pl.ds(base, C)], jj_v)
            cps = [
                pltpu.async_copy(rx_sp.at[ii_v], gix, sem),
                pltpu.async_copy(ry_sp.at[ii_v], giy, sem),
                pltpu.async_copy(rz_sp.at[ii_v], giz, sem),
                pltpu.async_copy(rx_sp.at[jj_v], gjx, sem),
                pltpu.async_copy(ry_sp.at[jj_v], gjy, sem),
                pltpu.async_copy(rz_sp.at[jj_v], gjz, sem),
            ]
            for cp in cps:
                cp.wait()
            pltpu.sync_copy(gix, ix_h.at[pl.ds(base, C)])
            pltpu.sync_copy(giy, iy_h.at[pl.ds(base, C)])
            pltpu.sync_copy(giz, iz_h.at[pl.ds(base, C)])
            pltpu.sync_copy(gjx, jx_h.at[pl.ds(base, C)])
            pltpu.sync_copy(gjy, jy_h.at[pl.ds(base, C)])
            pltpu.sync_copy(gjz, jz_h.at[pl.ds(base, C)])
            return carry

        lax.fori_loop(0, n_chunks, chunk_body, 0)

    return k(rx, ry, rz, idx_i, idx_j)


def _combine_body(ax_r, ay_r, az_r, bx_r, by_r, bz_r, ox_r, oy_r, oz_r,
                  cx_r, cy_r, cz_r):
    cx_r[...] = bx_r[...] - ax_r[...] + ox_r[...]
    cy_r[...] = by_r[...] - ay_r[...] + oy_r[...]
    cz_r[...] = bz_r[...] - az_r[...] + oz_r[...]


def _combine_tc(pi, pj, off):
    rows = pi[0].shape[0]
    br = 2000
    grid = rows // br
    spec = pl.BlockSpec((br, 128), lambda i: (i, 0))
    one = jax.ShapeDtypeStruct((rows, 128), jnp.float32)
    return pl.pallas_call(
        _combine_body,
        out_shape=(one, one, one),
        grid=(grid,),
        in_specs=[spec] * 9,
        out_specs=(spec, spec, spec),
    )(*pi, *pj, *off)


def kernel(R, offsets, idx_i, idx_j):
    E = idx_i.shape[0]
    rows = E // 128
    ix, iy, iz, jx, jy, jz = _gather_sc(
        R[:, 0], R[:, 1], R[:, 2],
        idx_i.astype(jnp.int32), idx_j.astype(jnp.int32))
    pi = [a.reshape(rows, 128) for a in (ix, iy, iz)]
    pj = [a.reshape(rows, 128) for a in (jx, jy, jz)]
    off = [offsets[:, c].reshape(rows, 128) for c in range(3)]
    cx, cy, cz = _combine_tc(pi, pj, off)
    return jnp.stack([cx.reshape(E), cy.reshape(E), cz.reshape(E)], axis=1)
